# baseline (speedup 1.0000x reference)
import functools

import jax
import jax.numpy as jnp
import numpy as np
from jax import lax
from jax.experimental import pallas as pl
from jax.experimental.pallas import tpu as pltpu
from jax.experimental.pallas import tpu_sc as plsc


def _selected_indices(k: int, n_bank: int) -> np.ndarray:
    centers = (np.arange(k, dtype=np.float32) + 0.5) * (n_bank / float(k))
    return np.clip(np.floor(centers).astype(np.int32), 0, n_bank - 1)


@functools.cache
def _make_sc_broadcast(B: int, N: int, D: int):
    info = plsc.get_sparse_core_info()
    nw = info.num_cores * info.num_subcores
    rows_per_w = N // nw
    mesh = plsc.VectorSubcoreMesh(core_axis_name="c", subcore_axis_name="s")

    n_chunks = 2
    rows_per_chunk = rows_per_w // n_chunks

    @functools.partial(
        pl.kernel,
        mesh=mesh,
        out_type=jax.ShapeDtypeStruct((B, N, D), jnp.float32),
        scratch_types=[
            pltpu.VMEM((rows_per_w, D), jnp.float32),
            pltpu.SemaphoreType.DMA,
            pltpu.SemaphoreType.DMA,
        ],
    )
    def sc_broadcast(bank_hbm, out_hbm, buf, rsem, wsem):
        wid = lax.axis_index("s") * info.num_cores + lax.axis_index("c")
        base = wid * rows_per_w
        reads = [
            pltpu.async_copy(
                bank_hbm.at[pl.ds(base, rows_per_chunk)],
                buf.at[pl.ds(0, rows_per_chunk)],
                rsem,
            )
        ]
        writes = []
        for c in range(n_chunks):
            reads[c].wait()
            if c + 1 < n_chunks:
                reads.append(
                    pltpu.async_copy(
                        bank_hbm.at[
                            pl.ds(base + (c + 1) * rows_per_chunk, rows_per_chunk)
                        ],
                        buf.at[pl.ds((c + 1) * rows_per_chunk, rows_per_chunk)],
                        rsem,
                    )
                )
            src = buf.at[pl.ds(c * rows_per_chunk, rows_per_chunk)]
            for b in range(B - 1):
                writes.append(
                    pltpu.async_copy(
                        src,
                        out_hbm.at[b].at[
                            pl.ds(base + c * rows_per_chunk, rows_per_chunk)
                        ],
                        wsem,
                    )
                )
            writes.append(
                pltpu.async_copy(
                    bank_hbm.at[pl.ds(base + c * rows_per_chunk, rows_per_chunk)],
                    out_hbm.at[B - 1].at[
                        pl.ds(base + c * rows_per_chunk, rows_per_chunk)
                    ],
                    wsem,
                )
            )
        for w in writes:
            w.wait()

    return sc_broadcast


def kernel(key_embed, bank):
    B = key_embed.shape[0]
    n_bank, dim = bank.shape
    k = min(2048, n_bank)
    idx = _selected_indices(k, n_bank)
    assert k == n_bank and np.array_equal(idx, np.arange(n_bank))
    q = _make_sc_broadcast(B, k, dim)(bank)
    q_valid = jnp.ones((B, k), dtype=jnp.bool_)
    return q, q_valid


# speedup vs baseline: 8.2047x; 8.2047x over previous
import functools

import jax
import jax.numpy as jnp
import numpy as np
from jax import lax
from jax.experimental import pallas as pl
from jax.experimental.pallas import tpu as pltpu
from jax.experimental.pallas import tpu_sc as plsc


def _selected_indices(k: int, n_bank: int) -> np.ndarray:
    centers = (np.arange(k, dtype=np.float32) + 0.5) * (n_bank / float(k))
    return np.clip(np.floor(centers).astype(np.int32), 0, n_bank - 1)


@functools.cache
def _make_sc_broadcast(B: int, N: int, D: int):
    info = plsc.get_sparse_core_info()
    nw = info.num_cores * info.num_subcores
    rows_per_w = N // nw
    mesh = plsc.VectorSubcoreMesh(core_axis_name="c", subcore_axis_name="s")

    @functools.partial(
        pl.kernel,
        mesh=mesh,
        out_type=jax.ShapeDtypeStruct((B, N, D), jnp.float32),
        scratch_types=[
            pltpu.VMEM((rows_per_w, D), jnp.float32),
            pltpu.SemaphoreType.DMA,
        ],
    )
    def sc_broadcast(bank_hbm, out_hbm, buf, sem):
        wid = lax.axis_index("s") * info.num_cores + lax.axis_index("c")
        base = wid * rows_per_w
        pltpu.sync_copy(bank_hbm.at[pl.ds(base, rows_per_w)], buf)
        writes = [
            pltpu.async_copy(buf, out_hbm.at[b].at[pl.ds(base, rows_per_w)], sem)
            for b in range(B)
        ]
        for w in writes:
            w.wait()

    return sc_broadcast


def kernel(key_embed, bank):
    B = key_embed.shape[0]
    n_bank, dim = bank.shape
    k = min(2048, n_bank)
    idx = _selected_indices(k, n_bank)
    assert k == n_bank and np.array_equal(idx, np.arange(n_bank))
    q = _make_sc_broadcast(B, k, dim)(bank)
    q_valid = jnp.ones((B, k), dtype=jnp.bool_)
    return q, q_valid
